# baseline (speedup 1.0000x reference)
import functools

import jax
import jax.numpy as jnp
from jax import lax
from jax.experimental import pallas as pl
from jax.experimental.pallas import tpu as pltpu
from jax.experimental.pallas import tpu_sc as plsc

N = 10000
E = 320000
D = 128
NUM_GRAPHS = 64

_NW = 32
_EDGES_PER_TILE = E // _NW
_K = 128
_CF = _EDGES_PER_TILE // _K
_KT = _EDGES_PER_TILE - _CF * _K
_NR = 3
_NI = 6
_UN = 6
_FULL = _CF // _UN
_RPT = 624
_RPT_LAST = N - 15 * _RPT


def _sc_agg_body(x_hbm, src_hbm, dst_hbm, out_hbm, rows,
                 s0, s1, s2, s3, s4, s5, d0, d1, d2, d3, d4, d5,
                 st, dt, acc, *sems):
    c = lax.axis_index("c")
    s = lax.axis_index("s")
    w = c * 16 + s
    r0 = s * _RPT
    sbufs = (s0, s1, s2, s3, s4, s5)
    dbufs = (d0, d1, d2, d3, d4, d5)
    semg = sems[:_NR]
    semi = sems[_NR:_NR + _NI]
    seminit = sems[_NR + _NI]
    semt = sems[_NR + _NI + 1]

    @pl.when(s < 15)
    def _():
        pltpu.async_copy(x_hbm.at[pl.ds(r0, _RPT)], acc.at[pl.ds(r0, _RPT)],
                         seminit)

    @pl.when(s == 15)
    def _():
        pltpu.async_copy(x_hbm.at[pl.ds(15 * _RPT, _RPT_LAST)],
                         acc.at[pl.ds(15 * _RPT, _RPT_LAST)], seminit)

    base = w * _EDGES_PER_TILE

    def issue_idx(chunk, j):
        pltpu.async_copy(src_hbm.at[pl.ds(base + chunk * _K, _K)],
                         sbufs[j], semi[j])
        pltpu.async_copy(dst_hbm.at[pl.ds(base + chunk * _K, _K)],
                         dbufs[j], semi[j])

    def wait_idx(j):
        pltpu.make_async_copy(src_hbm.at[pl.ds(0, _K)], sbufs[j],
                              semi[j]).wait()
        pltpu.make_async_copy(dst_hbm.at[pl.ds(0, _K)], dbufs[j],
                              semi[j]).wait()

    def issue_gather(j, b):
        pltpu.async_copy(x_hbm.at[sbufs[j]], rows.at[b], semg[b])

    def wait_gather(b):
        pltpu.make_async_copy(x_hbm.at[sbufs[0]], rows.at[b],
                              semg[b]).wait()

    for ch in range(_NR):
        issue_idx(ch, ch)
    pltpu.async_copy(src_hbm.at[pl.ds(base + _CF * _K, _KT)], st, semt)
    pltpu.async_copy(dst_hbm.at[pl.ds(base + _CF * _K, _KT)], dt, semt)
    for ch in range(_NR):
        wait_idx(ch)
        issue_gather(ch, ch)
    issue_idx(3, 3)
    issue_idx(4, 4)
    issue_idx(5, 5)

    @pl.when(s < 15)
    def _():
        pltpu.make_async_copy(x_hbm.at[pl.ds(r0, _RPT)],
                              acc.at[pl.ds(r0, _RPT)], seminit).wait()

    @pl.when(s == 15)
    def _():
        pltpu.make_async_copy(x_hbm.at[pl.ds(15 * _RPT, _RPT_LAST)],
                              acc.at[pl.ds(15 * _RPT, _RPT_LAST)],
                              seminit).wait()

    plsc.subcore_barrier()

    def body(i, carry):
        for u in range(_UN):
            cstep = i * _UN + u
            b = u % _NR
            j = u % _NI
            j3 = (u + _NR) % _NI
            wait_gather(b)
            pltpu.sync_copy(rows.at[b], acc.at[dbufs[j]], add=True)

            @pl.when(cstep + _NR < _CF)
            def _():
                wait_idx(j3)
                issue_gather(j3, b)

            @pl.when(cstep + _NI < _CF)
            def _():
                issue_idx(cstep + _NI, j)
        return carry

    lax.fori_loop(0, _FULL, body, 0)
    pltpu.make_async_copy(src_hbm.at[pl.ds(0, _KT)], st, semt).wait()
    pltpu.make_async_copy(dst_hbm.at[pl.ds(0, _KT)], dt, semt).wait()
    pltpu.async_copy(x_hbm.at[st], rows.at[0, pl.ds(0, _KT)], semg[0])
    pltpu.make_async_copy(x_hbm.at[st], rows.at[0, pl.ds(0, _KT)],
                          semg[0]).wait()
    pltpu.sync_copy(rows.at[0, pl.ds(0, _KT)], acc.at[dt], add=True)
    plsc.subcore_barrier()

    @pl.when(s < 15)
    def _():
        pltpu.sync_copy(acc.at[pl.ds(r0, _RPT)],
                        out_hbm.at[c, pl.ds(r0, _RPT)])

    @pl.when(s == 15)
    def _():
        pltpu.sync_copy(acc.at[pl.ds(15 * _RPT, _RPT_LAST)],
                        out_hbm.at[c, pl.ds(15 * _RPT, _RPT_LAST)])


_sc_agg = functools.partial(
    pl.kernel,
    out_type=jax.ShapeDtypeStruct((2, N, D), jnp.float32),
    mesh=plsc.VectorSubcoreMesh(core_axis_name="c", subcore_axis_name="s"),
    scratch_types=[
        pltpu.VMEM((_NR, _K, D), jnp.float32),
    ] + [pltpu.VMEM((_K,), jnp.int32)] * (2 * _NI) + [
        pltpu.VMEM((_KT,), jnp.int32),
        pltpu.VMEM((_KT,), jnp.int32),
        pltpu.VMEM_SHARED((N, D), jnp.float32),
    ] + [pltpu.SemaphoreType.DMA] * (_NR + _NI + 2),
)(_sc_agg_body)


_BM = 2000


def _bdot(a, b):
    return jax.lax.dot_general(
        a.astype(jnp.bfloat16), b.astype(jnp.bfloat16), (((1,), (0,)), ((), ())),
        preferred_element_type=jnp.float32)


def _mlp_kernel(p_ref, q_ref, x_ref, w1_ref, b1_ref, w2_ref, b2_ref, o_ref):
    h = p_ref[0] + q_ref[0] - x_ref[...]
    t = jnp.maximum(_bdot(h, w1_ref[...]) + b1_ref[...], 0.0)
    o_ref[...] = _bdot(t, w2_ref[...]) + b2_ref[...]


def _conv_mlp(parts, x, w1, b1, w2, b2):
    return pl.pallas_call(
        _mlp_kernel,
        grid=(N // _BM,),
        in_specs=[
            pl.BlockSpec((1, _BM, D), lambda i: (0, i, 0)),
            pl.BlockSpec((1, _BM, D), lambda i: (1, i, 0)),
            pl.BlockSpec((_BM, D), lambda i: (i, 0)),
            pl.BlockSpec((D, D), lambda i: (0, 0)),
            pl.BlockSpec((1, D), lambda i: (0, 0)),
            pl.BlockSpec((D, D), lambda i: (0, 0)),
            pl.BlockSpec((1, D), lambda i: (0, 0)),
        ],
        out_specs=pl.BlockSpec((_BM, D), lambda i: (i, 0)),
        out_shape=jax.ShapeDtypeStruct((N, D), jnp.float32),
    )(parts, parts, x, w1, b1.reshape(1, D), w2, b2.reshape(1, D))


_NF = N // _BM


def _fused_tail_kernel(b_ref, p_ref, q_ref, x1_ref, w1_ref, b1_ref, w2_ref,
                       b2_ref, wa_ref, wb_ref, bl1_ref, wl2_ref, bl2_ref,
                       o_ref, acc1, acc2):
    i = pl.program_id(0)
    h = p_ref[0] + q_ref[0] - x1_ref[...]
    t = jnp.maximum(_bdot(h, w1_ref[...]) + b1_ref[...], 0.0)
    x2 = _bdot(t, w2_ref[...]) + b2_ref[...]
    seg = b_ref[0, 0, :]
    iota = lax.broadcasted_iota(jnp.int32, (NUM_GRAPHS, _BM), 0)
    onehot = (seg[None, :] == iota).astype(jnp.float32)
    dn = (((1,), (0,)), ((), ()))
    d1 = lax.dot_general(onehot, x1_ref[...], dn,
                         preferred_element_type=jnp.float32)
    d2 = lax.dot_general(onehot, x2, dn, preferred_element_type=jnp.float32)

    @pl.when(i == 0)
    def _():
        acc1[...] = d1
        acc2[...] = d2

    @pl.when(i > 0)
    def _():
        acc1[...] += d1
        acc2[...] += d2

    @pl.when(i == _NF - 1)
    def _():
        hf = jnp.maximum(
            jnp.dot(acc1[...], wa_ref[...], preferred_element_type=jnp.float32)
            + jnp.dot(acc2[...], wb_ref[...], preferred_element_type=jnp.float32)
            + bl1_ref[...], 0.0)
        o_ref[...] = (
            jnp.dot(hf, wl2_ref[...], preferred_element_type=jnp.float32)
            + bl2_ref[...])


def _fused_tail(batch, parts, x1, w1, b1, w2, b2, wl1, bl1, wl2, bl2):
    full = pl.BlockSpec((D, D), lambda i: (0, 0))
    bias = pl.BlockSpec((1, D), lambda i: (0, 0))
    return pl.pallas_call(
        _fused_tail_kernel,
        grid=(_NF,),
        in_specs=[
            pl.BlockSpec((1, 1, _BM), lambda i: (i, 0, 0)),
            pl.BlockSpec((1, _BM, D), lambda i: (0, i, 0)),
            pl.BlockSpec((1, _BM, D), lambda i: (1, i, 0)),
            pl.BlockSpec((_BM, D), lambda i: (i, 0)),
            full, bias, full, bias,
            full, full, bias, full, bias,
        ],
        out_specs=pl.BlockSpec((NUM_GRAPHS, D), lambda i: (0, 0)),
        out_shape=jax.ShapeDtypeStruct((NUM_GRAPHS, D), jnp.float32),
        scratch_shapes=[
            pltpu.VMEM((NUM_GRAPHS, D), jnp.float32),
            pltpu.VMEM((NUM_GRAPHS, D), jnp.float32),
        ],
    )(batch.reshape(_NF, 1, _BM), parts, parts, x1,
      w1, b1.reshape(1, D), w2, b2.reshape(1, D),
      wl1[:D], wl1[D:], bl1.reshape(1, D), wl2, bl2.reshape(1, D))


def kernel(x, edge_index, batch,
           W0_1, b0_1, W0_2, b0_2,
           W1_1, b1_1, W1_2, b1_2,
           WL_1, bL_1, WL_2, bL_2):
    src = edge_index[0]
    dst = edge_index[1]
    p = _sc_agg(x, src, dst)
    x1 = _conv_mlp(p, x, W0_1, b0_1, W0_2, b0_2)
    p2 = _sc_agg(x1, src, dst)
    return _fused_tail(batch, p2, x1, W1_1, b1_1, W1_2, b1_2,
                       WL_1, bL_1, WL_2, bL_2)


# speedup vs baseline: 1.0292x; 1.0292x over previous
import functools

import jax
import jax.numpy as jnp
from jax import lax
from jax.experimental import pallas as pl
from jax.experimental.pallas import tpu as pltpu
from jax.experimental.pallas import tpu_sc as plsc

N = 10000
E = 320000
D = 128
NUM_GRAPHS = 64

_NW = 32
_EDGES_PER_TILE = E // _NW
_K = 80
_ITERS = _EDGES_PER_TILE // _K
_NR = 4
_NI = 6
_UN = 12
_FULL = 10
_RPT = 624
_RPT_LAST = N - 15 * _RPT


_ZR = 16


def _sc_agg_body(x_hbm, src_hbm, dst_hbm, out_hbm, rows,
                 s0, s1, s2, s3, s4, s5, d0, d1, d2, d3, d4, d5,
                 zbuf, acc, *sems):
    c = lax.axis_index("c")
    s = lax.axis_index("s")
    w = c * 16 + s
    r0 = s * _RPT
    sbufs = (s0, s1, s2, s3, s4, s5)
    dbufs = (d0, d1, d2, d3, d4, d5)
    semg = sems[:_NR]
    semi = sems[_NR:_NR + _NI]
    seminit = sems[_NR + _NI]
    zero16 = jnp.zeros((16,), jnp.float32)
    for zr in range(_ZR):
        for zl in range(D // 16):
            zbuf[zr, pl.ds(zl * 16, 16)] = zero16

    @pl.when(s < 15)
    def _():
        for k in range(_RPT // _ZR):
            pltpu.async_copy(zbuf, acc.at[pl.ds(r0 + k * _ZR, _ZR)], seminit)

    @pl.when(s == 15)
    def _():
        for k in range(_RPT_LAST // _ZR):
            pltpu.async_copy(zbuf, acc.at[pl.ds(15 * _RPT + k * _ZR, _ZR)],
                             seminit)

    base = w * _EDGES_PER_TILE

    def issue_idx(chunk, j):
        pltpu.async_copy(src_hbm.at[pl.ds(base + chunk * _K, _K)],
                         sbufs[j], semi[j])
        pltpu.async_copy(dst_hbm.at[pl.ds(base + chunk * _K, _K)],
                         dbufs[j], semi[j])

    def wait_idx(j):
        pltpu.make_async_copy(src_hbm.at[pl.ds(0, _K)], sbufs[j],
                              semi[j]).wait()
        pltpu.make_async_copy(dst_hbm.at[pl.ds(0, _K)], dbufs[j],
                              semi[j]).wait()

    def issue_gather(j, b):
        pltpu.async_copy(x_hbm.at[sbufs[j]], rows.at[b], semg[b])

    def wait_gather(b):
        pltpu.make_async_copy(x_hbm.at[sbufs[0]], rows.at[b],
                              semg[b]).wait()

    for ch in range(_NR):
        issue_idx(ch, ch)
    for ch in range(_NR):
        wait_idx(ch)
        issue_gather(ch, ch)
    issue_idx(4, 4)
    issue_idx(5, 5)

    @pl.when(s < 15)
    def _():
        for k in range(_RPT // _ZR):
            pltpu.make_async_copy(zbuf, acc.at[pl.ds(r0 + k * _ZR, _ZR)],
                                  seminit).wait()

    @pl.when(s == 15)
    def _():
        for k in range(_RPT_LAST // _ZR):
            pltpu.make_async_copy(zbuf,
                                  acc.at[pl.ds(15 * _RPT + k * _ZR, _ZR)],
                                  seminit).wait()

    plsc.subcore_barrier()

    def body(i, carry):
        for u in range(_UN):
            cstep = i * _UN + u
            b = u % _NR
            j = u % _NI
            j4 = (u + 4) % _NI
            wait_gather(b)
            pltpu.sync_copy(rows.at[b], acc.at[dbufs[j]], add=True)
            wait_idx(j4)
            issue_gather(j4, b)

            @pl.when(cstep + _NI < _ITERS)
            def _():
                issue_idx(cstep + _NI, j)
        return carry

    lax.fori_loop(0, _FULL, body, 0)
    for cstep in range(_FULL * _UN, _ITERS):
        b = cstep % _NR
        j = cstep % _NI
        wait_gather(b)
        pltpu.sync_copy(rows.at[b], acc.at[dbufs[j]], add=True)
        if cstep + _NR < _ITERS:
            j4 = (cstep + _NR) % _NI
            wait_idx(j4)
            issue_gather(j4, b)
    plsc.subcore_barrier()

    @pl.when(s < 15)
    def _():
        pltpu.sync_copy(acc.at[pl.ds(r0, _RPT)],
                        out_hbm.at[c, pl.ds(r0, _RPT)])

    @pl.when(s == 15)
    def _():
        pltpu.sync_copy(acc.at[pl.ds(15 * _RPT, _RPT_LAST)],
                        out_hbm.at[c, pl.ds(15 * _RPT, _RPT_LAST)])


_sc_agg = functools.partial(
    pl.kernel,
    out_type=jax.ShapeDtypeStruct((2, N, D), jnp.float32),
    mesh=plsc.VectorSubcoreMesh(core_axis_name="c", subcore_axis_name="s"),
    scratch_types=[
        pltpu.VMEM((_NR, _K, D), jnp.float32),
    ] + [pltpu.VMEM((_K,), jnp.int32)] * (2 * _NI) + [
        pltpu.VMEM((_ZR, D), jnp.float32),
        pltpu.VMEM_SHARED((N, D), jnp.float32),
    ] + [pltpu.SemaphoreType.DMA] * (_NR + _NI + 1),
)(_sc_agg_body)


_BM = 2000


def _bdot(a, b):
    return jax.lax.dot_general(
        a.astype(jnp.bfloat16), b.astype(jnp.bfloat16), (((1,), (0,)), ((), ())),
        preferred_element_type=jnp.float32)


def _mlp_kernel(p_ref, q_ref, x_ref, w1_ref, b1_ref, w2_ref, b2_ref, o_ref):
    h = p_ref[0] + q_ref[0] + x_ref[...]
    t = jnp.maximum(_bdot(h, w1_ref[...]) + b1_ref[...], 0.0)
    o_ref[...] = _bdot(t, w2_ref[...]) + b2_ref[...]


def _conv_mlp(parts, x, w1, b1, w2, b2):
    return pl.pallas_call(
        _mlp_kernel,
        grid=(N // _BM,),
        in_specs=[
            pl.BlockSpec((1, _BM, D), lambda i: (0, i, 0)),
            pl.BlockSpec((1, _BM, D), lambda i: (1, i, 0)),
            pl.BlockSpec((_BM, D), lambda i: (i, 0)),
            pl.BlockSpec((D, D), lambda i: (0, 0)),
            pl.BlockSpec((1, D), lambda i: (0, 0)),
            pl.BlockSpec((D, D), lambda i: (0, 0)),
            pl.BlockSpec((1, D), lambda i: (0, 0)),
        ],
        out_specs=pl.BlockSpec((_BM, D), lambda i: (i, 0)),
        out_shape=jax.ShapeDtypeStruct((N, D), jnp.float32),
    )(parts, parts, x, w1, b1.reshape(1, D), w2, b2.reshape(1, D))


_NF = N // _BM


def _fused_tail_kernel(b_ref, p_ref, q_ref, x1_ref, w1_ref, b1_ref, w2_ref,
                       b2_ref, wa_ref, wb_ref, bl1_ref, wl2_ref, bl2_ref,
                       o_ref, acc1, acc2):
    i = pl.program_id(0)
    h = p_ref[0] + q_ref[0] + x1_ref[...]
    t = jnp.maximum(_bdot(h, w1_ref[...]) + b1_ref[...], 0.0)
    x2 = _bdot(t, w2_ref[...]) + b2_ref[...]
    seg = b_ref[0, 0, :]
    iota = lax.broadcasted_iota(jnp.int32, (NUM_GRAPHS, _BM), 0)
    onehot = (seg[None, :] == iota).astype(jnp.float32)
    dn = (((1,), (0,)), ((), ()))
    d1 = lax.dot_general(onehot, x1_ref[...], dn,
                         preferred_element_type=jnp.float32)
    d2 = lax.dot_general(onehot, x2, dn, preferred_element_type=jnp.float32)

    @pl.when(i == 0)
    def _():
        acc1[...] = d1
        acc2[...] = d2

    @pl.when(i > 0)
    def _():
        acc1[...] += d1
        acc2[...] += d2

    @pl.when(i == _NF - 1)
    def _():
        hf = jnp.maximum(
            jnp.dot(acc1[...], wa_ref[...], preferred_element_type=jnp.float32)
            + jnp.dot(acc2[...], wb_ref[...], preferred_element_type=jnp.float32)
            + bl1_ref[...], 0.0)
        o_ref[...] = (
            jnp.dot(hf, wl2_ref[...], preferred_element_type=jnp.float32)
            + bl2_ref[...])


def _fused_tail(batch, parts, x1, w1, b1, w2, b2, wl1, bl1, wl2, bl2):
    full = pl.BlockSpec((D, D), lambda i: (0, 0))
    bias = pl.BlockSpec((1, D), lambda i: (0, 0))
    return pl.pallas_call(
        _fused_tail_kernel,
        grid=(_NF,),
        in_specs=[
            pl.BlockSpec((1, 1, _BM), lambda i: (i, 0, 0)),
            pl.BlockSpec((1, _BM, D), lambda i: (0, i, 0)),
            pl.BlockSpec((1, _BM, D), lambda i: (1, i, 0)),
            pl.BlockSpec((_BM, D), lambda i: (i, 0)),
            full, bias, full, bias,
            full, full, bias, full, bias,
        ],
        out_specs=pl.BlockSpec((NUM_GRAPHS, D), lambda i: (0, 0)),
        out_shape=jax.ShapeDtypeStruct((NUM_GRAPHS, D), jnp.float32),
        scratch_shapes=[
            pltpu.VMEM((NUM_GRAPHS, D), jnp.float32),
            pltpu.VMEM((NUM_GRAPHS, D), jnp.float32),
        ],
    )(batch.reshape(_NF, 1, _BM), parts, parts, x1,
      w1, b1.reshape(1, D), w2, b2.reshape(1, D),
      wl1[:D], wl1[D:], bl1.reshape(1, D), wl2, bl2.reshape(1, D))


def kernel(x, edge_index, batch,
           W0_1, b0_1, W0_2, b0_2,
           W1_1, b1_1, W1_2, b1_2,
           WL_1, bL_1, WL_2, bL_2):
    src = edge_index[0]
    dst = edge_index[1]
    p = _sc_agg(x, src, dst)
    x1 = _conv_mlp(p, x, W0_1, b0_1, W0_2, b0_2)
    p2 = _sc_agg(x1, src, dst)
    return _fused_tail(batch, p2, x1, W1_1, b1_1, W1_2, b1_2,
                       WL_1, bL_1, WL_2, bL_2)
